# Initial kernel scaffold; baseline (speedup 1.0000x reference)
import jax
import jax.numpy as jnp
from jax.experimental import pallas as pl


def kernel(x, edge_index, W1l, b1l, W1r, W2l, b2l, W2r):
    raise NotImplementedError("write your pallas kernel here")


# speedup vs baseline: 3.5025x; 3.5025x over previous
import functools

import jax
import jax.numpy as jnp
from jax import lax
from jax.experimental import pallas as pl
from jax.experimental.pallas import tpu as pltpu
from jax.experimental.pallas import tpu_sc as plsc

N = 10000
E = 320000
D_IN = 128
D_OUT = 128
D_HID = 256
SLOPE = 0.01

N_PAD = 10240
E_PAD = 327680

NUM_CORES = 2
NUM_SUBCORES = 16
NUM_TILES = NUM_CORES * NUM_SUBCORES
CHUNK = 128
CHUNKS_PER_TILE = E_PAD // CHUNK // NUM_TILES
ROWS_PER_TILE = N_PAD // NUM_SUBCORES


def _sc_segment_sum(table, src_r, dst_r, zeros_blk, zeros_row, with_counts):
  mesh = plsc.VectorSubcoreMesh(core_axis_name="c", subcore_axis_name="s")
  f32 = jnp.float32

  out_type = [jax.ShapeDtypeStruct((NUM_CORES, N_PAD, D_IN), f32)]
  scratch = [
      pltpu.VMEM((CHUNKS_PER_TILE, CHUNK), jnp.int32),
      pltpu.VMEM((CHUNKS_PER_TILE, CHUNK), jnp.int32),
      pltpu.VMEM((CHUNK, D_IN), f32),
      pltpu.VMEM_SHARED((N_PAD, D_IN), f32),
      pltpu.SemaphoreType.DMA,
  ]
  if with_counts:
    out_type.append(jax.ShapeDtypeStruct((NUM_TILES * N_PAD,), f32))
    scratch.append(pltpu.VMEM((N_PAD,), f32))

  @functools.partial(
      pl.kernel, mesh=mesh, out_type=out_type, scratch_types=scratch,
      compiler_params=pltpu.CompilerParams(needs_layout_passes=False))
  def k(table_hbm, src_hbm, dst_hbm, zeros_hbm, zrow_hbm, *refs):
    if with_counts:
      out_hbm, cnt_hbm, src_v, dst_v, rows_v, accum, sem, cnt_local = refs
    else:
      out_hbm, src_v, dst_v, rows_v, accum, sem = refs
    c = lax.axis_index("c")
    s = lax.axis_index("s")
    wid = c * NUM_SUBCORES + s

    base_r = s * ROWS_PER_TILE
    for j in range(ROWS_PER_TILE // CHUNK):
      pltpu.sync_copy(zeros_hbm, accum.at[pl.ds(base_r + j * CHUNK, CHUNK)])
    if with_counts:
      pltpu.sync_copy(zrow_hbm, cnt_local)

    base_c = wid * CHUNKS_PER_TILE
    pltpu.sync_copy(src_hbm.at[pl.ds(base_c, CHUNKS_PER_TILE)], src_v)
    pltpu.sync_copy(dst_hbm.at[pl.ds(base_c, CHUNKS_PER_TILE)], dst_v)
    plsc.subcore_barrier()

    ones16 = jnp.ones((16,), f32)

    def ebody(j, carry):
      gcopy = pltpu.async_copy(table_hbm.at[src_v.at[j]], rows_v, sem)
      if with_counts:
        for kk in range(CHUNK // 16):
          d16 = dst_v[j, pl.ds(kk * 16, 16)]
          plsc.addupdate_scatter(cnt_local, [d16], ones16)
      gcopy.wait()
      pltpu.sync_copy(rows_v, accum.at[dst_v.at[j]], add=True)
      return carry

    lax.fori_loop(0, CHUNKS_PER_TILE, ebody, 0)
    if with_counts:
      pltpu.sync_copy(cnt_local, cnt_hbm.at[pl.ds(wid * N_PAD, N_PAD)])
    plsc.subcore_barrier()

    pltpu.sync_copy(accum.at[pl.ds(base_r, ROWS_PER_TILE)],
                    out_hbm.at[c].at[pl.ds(base_r, ROWS_PER_TILE)])

  return k(table, src_r, dst_r, zeros_blk, zeros_row)


_DN = (((1,), (1,)), ((), ()))
_R = 256


def _tc1_body(ps, cc, x, w1l, b1l, w1r, w2l, w2r, g_ref, r_ref, inv_ref):
  srows = ps[0] + ps[1]
  cnt = jnp.sum(cc[...], axis=0)
  inv = 1.0 / jnp.maximum(cnt, 1.0)
  t = lax.dot_general(srows, w1l[...], _DN,
                      preferred_element_type=jnp.float32) * inv
  t = t + b1l[...] + lax.dot_general(x[...], w1r[...], _DN,
                                     preferred_element_type=jnp.float32)
  h = jnp.where(t > 0, t, SLOPE * t)
  g_ref[...] = lax.dot_general(h, w2l[...], _DN,
                               preferred_element_type=jnp.float32)
  r_ref[...] = lax.dot_general(h, w2r[...], _DN,
                               preferred_element_type=jnp.float32)
  inv_ref[...] = jnp.broadcast_to(inv, (_R, D_OUT))


def _tc2_body(q, inv, r, b2l, out_ref):
  out_ref[...] = (q[0] + q[1]) * inv[...] + b2l[...] + r[...]


def kernel(x, edge_index, W1l, b1l, W1r, W2l, b2l, W2r):
  f32 = jnp.float32
  x_p = jnp.zeros((N_PAD, D_IN), f32).at[:N].set(x)

  src = jnp.concatenate(
      [edge_index[0], jnp.zeros((E_PAD - E,), jnp.int32)]).reshape(-1, CHUNK)
  dst = jnp.concatenate(
      [edge_index[1],
       jnp.full((E_PAD - E,), N_PAD - 1, jnp.int32)]).reshape(-1, CHUNK)

  zeros_blk = jnp.zeros((CHUNK, 128), f32)
  zeros_row = jnp.zeros((N_PAD,), f32)
  b1l_r = b1l.reshape(1, D_HID)
  b2l_r = b2l.reshape(1, D_OUT)

  p1, cnt = _sc_segment_sum(x_p, src, dst, zeros_blk, zeros_row, True)
  cnt_col = cnt.reshape(NUM_TILES, N_PAD, 1)

  grid = (N_PAD // _R,)
  g, r, invb = pl.pallas_call(
      _tc1_body,
      grid=grid,
      in_specs=[
          pl.BlockSpec((NUM_CORES, _R, D_IN), lambda i: (0, i, 0)),
          pl.BlockSpec((NUM_TILES, _R, 1), lambda i: (0, i, 0)),
          pl.BlockSpec((_R, D_IN), lambda i: (i, 0)),
          pl.BlockSpec((D_HID, D_IN), lambda i: (0, 0)),
          pl.BlockSpec((1, D_HID), lambda i: (0, 0)),
          pl.BlockSpec((D_HID, D_IN), lambda i: (0, 0)),
          pl.BlockSpec((D_OUT, D_HID), lambda i: (0, 0)),
          pl.BlockSpec((D_OUT, D_HID), lambda i: (0, 0)),
      ],
      out_specs=[
          pl.BlockSpec((_R, D_OUT), lambda i: (i, 0)),
          pl.BlockSpec((_R, D_OUT), lambda i: (i, 0)),
          pl.BlockSpec((_R, D_OUT), lambda i: (i, 0)),
      ],
      out_shape=[
          jax.ShapeDtypeStruct((N_PAD, D_OUT), f32),
          jax.ShapeDtypeStruct((N_PAD, D_OUT), f32),
          jax.ShapeDtypeStruct((N_PAD, D_OUT), f32),
      ],
  )(p1, cnt_col, x_p, W1l, b1l_r, W1r, W2l, W2r)

  p2 = _sc_segment_sum(g, src, dst, zeros_blk, zeros_row, False)
  if isinstance(p2, (list, tuple)):
    (p2,) = p2

  out_full = pl.pallas_call(
      _tc2_body,
      grid=grid,
      in_specs=[
          pl.BlockSpec((NUM_CORES, _R, D_OUT), lambda i: (0, i, 0)),
          pl.BlockSpec((_R, D_OUT), lambda i: (i, 0)),
          pl.BlockSpec((_R, D_OUT), lambda i: (i, 0)),
          pl.BlockSpec((1, D_OUT), lambda i: (0, 0)),
      ],
      out_specs=pl.BlockSpec((_R, D_OUT), lambda i: (i, 0)),
      out_shape=jax.ShapeDtypeStruct((N_PAD, D_OUT), f32),
  )(p2, invb, r, b2l_r)

  return out_full[:N]


# speedup vs baseline: 3.9031x; 1.1144x over previous
import functools

import jax
import jax.numpy as jnp
from jax import lax
from jax.experimental import pallas as pl
from jax.experimental.pallas import tpu as pltpu
from jax.experimental.pallas import tpu_sc as plsc

N = 10000
E = 320000
D_IN = 128
D_OUT = 128
D_HID = 256
SLOPE = 0.01

N_PAD = 10240
E_PAD = 327680

NUM_CORES = 2
NUM_SUBCORES = 16
NUM_TILES = NUM_CORES * NUM_SUBCORES
CHUNK = 128
CHUNKS_PER_TILE = E_PAD // CHUNK // NUM_TILES
ROWS_PER_TILE = N_PAD // NUM_SUBCORES
ZBLK = 128
BLK = 8
NBLK = CHUNKS_PER_TILE // BLK


def _sc_segment_sum(table, comb, zeros_blk, zeros_row, with_counts):
  mesh = plsc.VectorSubcoreMesh(core_axis_name="c", subcore_axis_name="s")
  f32 = jnp.float32

  out_type = [jax.ShapeDtypeStruct((NUM_CORES, N_PAD, D_IN), f32)]
  scratch = [
      pltpu.VMEM((2, BLK, 2, CHUNK), jnp.int32),
      pltpu.VMEM((CHUNK, D_IN), f32),
      pltpu.VMEM((CHUNK, D_IN), f32),
      pltpu.VMEM_SHARED((N_PAD, D_IN), f32),
      pltpu.SemaphoreType.DMA,
      pltpu.SemaphoreType.DMA,
  ]
  if with_counts:
    out_type.append(jax.ShapeDtypeStruct((NUM_TILES * N_PAD,), f32))
    scratch.append(pltpu.VMEM((N_PAD,), f32))

  @functools.partial(
      pl.kernel, mesh=mesh, out_type=out_type, scratch_types=scratch,
      compiler_params=pltpu.CompilerParams(needs_layout_passes=False))
  def k(table_hbm, comb_hbm, zeros_hbm, zrow_hbm, *refs):
    if with_counts:
      out_hbm, cnt_hbm, ring, rows_a, rows_b, accum, sem_a, sem_b, \
          cnt_local = refs
    else:
      out_hbm, ring, rows_a, rows_b, accum, sem_a, sem_b = refs
    c = lax.axis_index("c")
    s = lax.axis_index("s")
    wid = c * NUM_SUBCORES + s

    base_r = s * ROWS_PER_TILE
    for j in range(ROWS_PER_TILE // ZBLK):
      pltpu.sync_copy(zeros_hbm, accum.at[pl.ds(base_r + j * ZBLK, ZBLK)])
    if with_counts:
      pltpu.sync_copy(zrow_hbm, cnt_local)

    ones16 = jnp.ones((16,), f32)
    base_c = wid * CHUNKS_PER_TILE

    pltpu.sync_copy(comb_hbm.at[pl.ds(base_c, BLK)], ring.at[0])
    pltpu.async_copy(table_hbm.at[ring.at[0, 0, 0]], rows_a, sem_a)
    pltpu.async_copy(table_hbm.at[ring.at[0, 1, 0]], rows_b, sem_b)
    plsc.subcore_barrier()

    def bbody(b, carry):
      par = b % 2

      @pl.when(b + 1 < NBLK)
      def _():
        pltpu.sync_copy(comb_hbm.at[pl.ds(base_c + (b + 1) * BLK, BLK)],
                        ring.at[1 - par])
      for r in range(BLK):
        rows_v, sem = (rows_a, sem_a) if r % 2 == 0 else (rows_b, sem_b)
        if with_counts:
          for kk in range(CHUNK // 16):
            d16 = ring[par, r, 1, pl.ds(kk * 16, 16)]
            plsc.addupdate_scatter(cnt_local, [d16], ones16)
        pltpu.make_async_copy(table_hbm.at[ring.at[par, r, 0]],
                              rows_v, sem).wait()
        pltpu.sync_copy(rows_v, accum.at[ring.at[par, r, 1]], add=True)
        if r + 2 < BLK:
          pltpu.async_copy(table_hbm.at[ring.at[par, r + 2, 0]], rows_v, sem)
        else:
          @pl.when(b + 1 < NBLK)
          def _():
            pltpu.async_copy(table_hbm.at[ring.at[1 - par, r + 2 - BLK, 0]],
                             rows_v, sem)
      return carry

    lax.fori_loop(0, NBLK, bbody, 0)
    if with_counts:
      pltpu.sync_copy(cnt_local, cnt_hbm.at[pl.ds(wid * N_PAD, N_PAD)])
    plsc.subcore_barrier()

    pltpu.sync_copy(accum.at[pl.ds(base_r, ROWS_PER_TILE)],
                    out_hbm.at[c].at[pl.ds(base_r, ROWS_PER_TILE)])

  return k(table, comb, zeros_blk, zeros_row)


_DN = (((1,), (1,)), ((), ()))
_R = 256


def _tc1_body(ps, cc, x, w1l, b1l, w1r, w2l, w2r, g_ref, r_ref, inv_ref):
  srows = ps[0] + ps[1]
  cnt = jnp.sum(cc[...], axis=0)
  inv = 1.0 / jnp.maximum(cnt, 1.0)
  t = lax.dot_general(srows, w1l[...], _DN,
                      preferred_element_type=jnp.float32) * inv
  t = t + b1l[...] + lax.dot_general(x[...], w1r[...], _DN,
                                     preferred_element_type=jnp.float32)
  h = jnp.where(t > 0, t, SLOPE * t)
  g_ref[...] = lax.dot_general(h, w2l[...], _DN,
                               preferred_element_type=jnp.float32)
  r_ref[...] = lax.dot_general(h, w2r[...], _DN,
                               preferred_element_type=jnp.float32)
  inv_ref[...] = jnp.broadcast_to(inv, (_R, D_OUT))


def _tc2_body(q, inv, r, b2l, out_ref):
  out_ref[...] = (q[0] + q[1]) * inv[...] + b2l[...] + r[...]


def kernel(x, edge_index, W1l, b1l, W1r, W2l, b2l, W2r):
  f32 = jnp.float32
  x_p = jnp.zeros((N_PAD, D_IN), f32).at[:N].set(x)

  src = jnp.concatenate(
      [edge_index[0], jnp.zeros((E_PAD - E,), jnp.int32)]).reshape(-1, CHUNK)
  pad_dst = N + jnp.arange(E_PAD - E, dtype=jnp.int32) % (N_PAD - N)
  dst = jnp.concatenate([edge_index[1], pad_dst]).reshape(-1, CHUNK)
  comb = jnp.stack([src, dst], axis=1)

  zeros_blk = jnp.zeros((ZBLK, 128), f32)
  zeros_row = jnp.zeros((N_PAD,), f32)
  b1l_r = b1l.reshape(1, D_HID)
  b2l_r = b2l.reshape(1, D_OUT)

  p1, cnt = _sc_segment_sum(x_p, comb, zeros_blk, zeros_row, True)
  cnt_col = cnt.reshape(NUM_TILES, N_PAD, 1)

  grid = (N_PAD // _R,)
  g, r, invb = pl.pallas_call(
      _tc1_body,
      grid=grid,
      in_specs=[
          pl.BlockSpec((NUM_CORES, _R, D_IN), lambda i: (0, i, 0)),
          pl.BlockSpec((NUM_TILES, _R, 1), lambda i: (0, i, 0)),
          pl.BlockSpec((_R, D_IN), lambda i: (i, 0)),
          pl.BlockSpec((D_HID, D_IN), lambda i: (0, 0)),
          pl.BlockSpec((1, D_HID), lambda i: (0, 0)),
          pl.BlockSpec((D_HID, D_IN), lambda i: (0, 0)),
          pl.BlockSpec((D_OUT, D_HID), lambda i: (0, 0)),
          pl.BlockSpec((D_OUT, D_HID), lambda i: (0, 0)),
      ],
      out_specs=[
          pl.BlockSpec((_R, D_OUT), lambda i: (i, 0)),
          pl.BlockSpec((_R, D_OUT), lambda i: (i, 0)),
          pl.BlockSpec((_R, D_OUT), lambda i: (i, 0)),
      ],
      out_shape=[
          jax.ShapeDtypeStruct((N_PAD, D_OUT), f32),
          jax.ShapeDtypeStruct((N_PAD, D_OUT), f32),
          jax.ShapeDtypeStruct((N_PAD, D_OUT), f32),
      ],
  )(p1, cnt_col, x_p, W1l, b1l_r, W1r, W2l, W2r)

  p2 = _sc_segment_sum(g, comb, zeros_blk, zeros_row, False)
  if isinstance(p2, (list, tuple)):
    (p2,) = p2

  out_full = pl.pallas_call(
      _tc2_body,
      grid=grid,
      in_specs=[
          pl.BlockSpec((NUM_CORES, _R, D_OUT), lambda i: (0, i, 0)),
          pl.BlockSpec((_R, D_OUT), lambda i: (i, 0)),
          pl.BlockSpec((_R, D_OUT), lambda i: (i, 0)),
          pl.BlockSpec((1, D_OUT), lambda i: (0, 0)),
      ],
      out_specs=pl.BlockSpec((_R, D_OUT), lambda i: (i, 0)),
      out_shape=jax.ShapeDtypeStruct((N_PAD, D_OUT), f32),
  )(p2, invb, r, b2l_r)

  return out_full[:N]


# speedup vs baseline: 4.1913x; 1.0738x over previous
import functools

import jax
import jax.numpy as jnp
from jax import lax
from jax.experimental import pallas as pl
from jax.experimental.pallas import tpu as pltpu
from jax.experimental.pallas import tpu_sc as plsc

N = 10000
E = 320000
D_IN = 128
D_OUT = 128
D_HID = 256
SLOPE = 0.01

N_PAD = 10240
E_PAD = 327680

NUM_CORES = 2
NUM_SUBCORES = 16
NUM_TILES = NUM_CORES * NUM_SUBCORES
CHUNK = 128
CHUNKS_PER_TILE = E_PAD // CHUNK // NUM_TILES
ROWS_PER_TILE = N_PAD // NUM_SUBCORES
ZBLK = 128
BLK = 8
NBLK = CHUNKS_PER_TILE // BLK


def _sc_segment_sum(table, comb, zeros_blk, zeros_row, with_counts):
  mesh = plsc.VectorSubcoreMesh(core_axis_name="c", subcore_axis_name="s")
  f32 = jnp.float32

  out_type = [jax.ShapeDtypeStruct((NUM_CORES, N_PAD, D_IN), f32)]
  scratch = [
      pltpu.VMEM((2, BLK, 2, CHUNK), jnp.int32),
      pltpu.VMEM((CHUNK, D_IN), f32),
      pltpu.VMEM((CHUNK, D_IN), f32),
      pltpu.VMEM_SHARED((N_PAD, D_IN), f32),
      pltpu.SemaphoreType.DMA,
      pltpu.SemaphoreType.DMA,
  ]
  if with_counts:
    out_type.append(jax.ShapeDtypeStruct((NUM_TILES * N_PAD,), f32))
    scratch.append(pltpu.VMEM((N_PAD,), f32))

  @functools.partial(
      pl.kernel, mesh=mesh, out_type=out_type, scratch_types=scratch,
      compiler_params=pltpu.CompilerParams(needs_layout_passes=False))
  def k(table_hbm, comb_hbm, zeros_hbm, zrow_hbm, *refs):
    if with_counts:
      out_hbm, cnt_hbm, ring, rows_a, rows_b, accum, sem_a, sem_b, \
          cnt_local = refs
    else:
      out_hbm, ring, rows_a, rows_b, accum, sem_a, sem_b = refs
    c = lax.axis_index("c")
    s = lax.axis_index("s")
    wid = c * NUM_SUBCORES + s

    base_r = s * ROWS_PER_TILE
    for j in range(ROWS_PER_TILE // ZBLK):
      pltpu.sync_copy(zeros_hbm, accum.at[pl.ds(base_r + j * ZBLK, ZBLK)])
    if with_counts:
      pltpu.sync_copy(zrow_hbm, cnt_local)

    ones16 = jnp.ones((16,), f32)
    base_c = wid * CHUNKS_PER_TILE

    pltpu.sync_copy(comb_hbm.at[pl.ds(base_c, BLK)], ring.at[0])
    pltpu.async_copy(table_hbm.at[ring.at[0, 0, 0]], rows_a, sem_a)
    pltpu.async_copy(table_hbm.at[ring.at[0, 1, 0]], rows_b, sem_b)
    plsc.subcore_barrier()

    def bbody(b, carry):
      par = b % 2

      @pl.when(b + 1 < NBLK)
      def _():
        pltpu.sync_copy(comb_hbm.at[pl.ds(base_c + (b + 1) * BLK, BLK)],
                        ring.at[1 - par])
      for r in range(BLK):
        rows_v, sem = (rows_a, sem_a) if r % 2 == 0 else (rows_b, sem_b)
        if with_counts:
          for kk in range(CHUNK // 16):
            d16 = ring[par, r, 1, pl.ds(kk * 16, 16)]
            plsc.addupdate_scatter(cnt_local, [d16], ones16)
        pltpu.make_async_copy(table_hbm.at[ring.at[par, r, 0]],
                              rows_v, sem).wait()
        pltpu.sync_copy(rows_v, accum.at[ring.at[par, r, 1]], add=True)
        if r + 2 < BLK:
          pltpu.async_copy(table_hbm.at[ring.at[par, r + 2, 0]], rows_v, sem)
        else:
          @pl.when(b + 1 < NBLK)
          def _():
            pltpu.async_copy(table_hbm.at[ring.at[1 - par, r + 2 - BLK, 0]],
                             rows_v, sem)
      return carry

    lax.fori_loop(0, NBLK, bbody, 0)
    if with_counts:
      pltpu.sync_copy(cnt_local, cnt_hbm.at[pl.ds(wid * N_PAD, N_PAD)])
    plsc.subcore_barrier()

    pltpu.sync_copy(accum.at[pl.ds(base_r, ROWS_PER_TILE)],
                    out_hbm.at[c].at[pl.ds(base_r, ROWS_PER_TILE)])

  return k(table, comb, zeros_blk, zeros_row)


_DN = (((1,), (1,)), ((), ()))
_R = 1024


def _tc1_body(ps, cc, x, w1l, b1l, w1r, w2l, w2r, g_ref, r_ref, inv_ref):
  srows = ps[0] + ps[1]
  cnt2 = jnp.sum(cc[...], axis=0)
  f32 = jnp.float32
  i32 = jnp.int32
  rows8 = lax.broadcasted_iota(i32, (_R, _R // 128), 0) // 128
  cols8 = lax.broadcasted_iota(i32, (_R, _R // 128), 1)
  expand = (rows8 == cols8).astype(f32)
  lane_r = lax.broadcasted_iota(i32, (_R, 128), 0) % 128
  lane_c = lax.broadcasted_iota(i32, (_R, 128), 1)
  sel = (lane_r == lane_c).astype(f32)
  rep = lax.dot_general(expand, cnt2, (((1,), (0,)), ((), ())),
                        preferred_element_type=f32)
  cnt = jnp.sum(rep * sel, axis=1, keepdims=True)
  inv = 1.0 / jnp.maximum(cnt, 1.0)
  t = lax.dot_general(srows, w1l[...], _DN,
                      preferred_element_type=jnp.float32) * inv
  t = t + b1l[...] + lax.dot_general(x[...], w1r[...], _DN,
                                     preferred_element_type=jnp.float32)
  h = jnp.where(t > 0, t, SLOPE * t)
  g_ref[...] = lax.dot_general(h, w2l[...], _DN,
                               preferred_element_type=jnp.float32)
  r_ref[...] = lax.dot_general(h, w2r[...], _DN,
                               preferred_element_type=jnp.float32)
  inv_ref[...] = jnp.broadcast_to(inv, (_R, D_OUT))


def _tc2_body(q, inv, r, b2l, out_ref):
  out_ref[...] = (q[0] + q[1]) * inv[...] + b2l[...] + r[...]


def kernel(x, edge_index, W1l, b1l, W1r, W2l, b2l, W2r):
  f32 = jnp.float32
  x_p = jnp.zeros((N_PAD, D_IN), f32).at[:N].set(x)

  src = jnp.concatenate(
      [edge_index[0], jnp.zeros((E_PAD - E,), jnp.int32)]).reshape(-1, CHUNK)
  pad_dst = N + jnp.arange(E_PAD - E, dtype=jnp.int32) % (N_PAD - N)
  dst = jnp.concatenate([edge_index[1], pad_dst]).reshape(-1, CHUNK)
  comb = jnp.stack([src, dst], axis=1)

  zeros_blk = jnp.zeros((ZBLK, 128), f32)
  zeros_row = jnp.zeros((N_PAD,), f32)
  b1l_r = b1l.reshape(1, D_HID)
  b2l_r = b2l.reshape(1, D_OUT)

  p1, cnt = _sc_segment_sum(x_p, comb, zeros_blk, zeros_row, True)
  cnt_col = cnt.reshape(NUM_TILES, N_PAD // 128, 128)

  grid = (N_PAD // _R,)
  g, r, invb = pl.pallas_call(
      _tc1_body,
      grid=grid,
      in_specs=[
          pl.BlockSpec((NUM_CORES, _R, D_IN), lambda i: (0, i, 0)),
          pl.BlockSpec((NUM_TILES, _R // 128, 128), lambda i: (0, i, 0)),
          pl.BlockSpec((_R, D_IN), lambda i: (i, 0)),
          pl.BlockSpec((D_HID, D_IN), lambda i: (0, 0)),
          pl.BlockSpec((1, D_HID), lambda i: (0, 0)),
          pl.BlockSpec((D_HID, D_IN), lambda i: (0, 0)),
          pl.BlockSpec((D_OUT, D_HID), lambda i: (0, 0)),
          pl.BlockSpec((D_OUT, D_HID), lambda i: (0, 0)),
      ],
      out_specs=[
          pl.BlockSpec((_R, D_OUT), lambda i: (i, 0)),
          pl.BlockSpec((_R, D_OUT), lambda i: (i, 0)),
          pl.BlockSpec((_R, D_OUT), lambda i: (i, 0)),
      ],
      out_shape=[
          jax.ShapeDtypeStruct((N_PAD, D_OUT), f32),
          jax.ShapeDtypeStruct((N_PAD, D_OUT), f32),
          jax.ShapeDtypeStruct((N_PAD, D_OUT), f32),
      ],
  )(p1, cnt_col, x_p, W1l, b1l_r, W1r, W2l, W2r)

  p2 = _sc_segment_sum(g, comb, zeros_blk, zeros_row, False)
  if isinstance(p2, (list, tuple)):
    (p2,) = p2

  out_full = pl.pallas_call(
      _tc2_body,
      grid=grid,
      in_specs=[
          pl.BlockSpec((NUM_CORES, _R, D_OUT), lambda i: (0, i, 0)),
          pl.BlockSpec((_R, D_OUT), lambda i: (i, 0)),
          pl.BlockSpec((_R, D_OUT), lambda i: (i, 0)),
          pl.BlockSpec((1, D_OUT), lambda i: (0, 0)),
      ],
      out_specs=pl.BlockSpec((_R, D_OUT), lambda i: (i, 0)),
      out_shape=jax.ShapeDtypeStruct((N_PAD, D_OUT), f32),
  )(p2, invb, r, b2l_r)

  return out_full[:N]
